# baseline (speedup 1.0000x reference)
import functools

import jax
import jax.numpy as jnp
from jax import lax
from jax.experimental import pallas as pl
from jax.experimental.pallas import tpu as pltpu
from jax.experimental.pallas import tpu_sc as plsc

NC = 2
NS = 16
L = 16
NW = NC * NS


@functools.lru_cache(maxsize=None)
def _build(B, ITEM_N, USER_N, M, D, C):
    per_w = B // NW
    n_chunks = per_w // C
    n_groups = C // L
    JMAX = M - 2

    mesh = plsc.VectorSubcoreMesh(core_axis_name="c", subcore_axis_name="s")

    @functools.partial(
        pl.kernel,
        out_type=jax.ShapeDtypeStruct((B,), jnp.float32),
        mesh=mesh,
        compiler_params=pltpu.CompilerParams(
            needs_layout_passes=False, use_tc_tiling_on_sc=False),
        scratch_types=[
            pltpu.VMEM((C,), jnp.int32),
            pltpu.VMEM((C,), jnp.int32),
            pltpu.VMEM((C, M * D), jnp.float32),
            pltpu.VMEM((C, D), jnp.float32),
            pltpu.VMEM((C,), jnp.float32),
            pltpu.VMEM((C,), jnp.float32),
            pltpu.SemaphoreType.DMA,
        ],
    )
    def kern(uid_hbm, iid_hbm, users_hbm, items_hbm, nb_hbm, out_hbm,
             iid_v, uid_v, e_v, u_v, nb_v, resp_v, sem):
        wid = lax.axis_index("s") * NC + lax.axis_index("c")
        base = wid * per_w

        def chunk_body(c, carry):
            off = base + c * C
            pltpu.sync_copy(iid_hbm.at[pl.ds(off, C)], iid_v)
            pltpu.sync_copy(uid_hbm.at[pl.ds(off, C)], uid_v)
            h1 = pltpu.async_copy(items_hbm.at[iid_v], e_v, sem)
            h2 = pltpu.async_copy(users_hbm.at[uid_v], u_v, sem)
            h3 = pltpu.async_copy(nb_hbm.at[iid_v], nb_v, sem)
            h1.wait()
            h2.wait()
            h3.wait()

            def group_body(g, carry2):
                rows = g * L + lax.iota(jnp.int32, L)
                nb_f = nb_v[pl.ds(g * L, L)]

                accs0 = tuple(jnp.zeros((L,), jnp.float32)
                              for _ in range(JMAX))

                def d_body(dd, accs):
                    dvec = jnp.full((L,), dd, jnp.int32)
                    u_val = plsc.load_gather(u_v, [rows, dvec])
                    new = []
                    for j in range(JMAX):
                        col = dvec + (j + 1) * D
                        e_val = plsc.load_gather(e_v, [rows, col])
                        diff = u_val - e_val
                        new.append(accs[j] + diff * diff)
                    return tuple(new)

                accs = lax.fori_loop(0, D, d_body, accs0)

                inf = jnp.full((L,), jnp.inf, jnp.float32)
                best = inf
                bj = jnp.zeros((L,), jnp.float32)
                for j in range(JMAX):
                    jj = float(j + 1)
                    dj = jnp.where(nb_f >= jj, accs[j], inf)
                    upd = dj < best
                    best = jnp.where(upd, dj, best)
                    bj = jnp.where(upd, jj, bj)
                resp = (bj - 1.0) / (nb_f - 1.0) + 1.0
                resp_v[pl.ds(g * L, L)] = resp
                return carry2

            lax.fori_loop(0, n_groups, group_body, 0)
            pltpu.sync_copy(resp_v, out_hbm.at[pl.ds(off, C)])
            return carry

        lax.fori_loop(0, n_chunks, chunk_body, 0)

    return kern


def kernel(user_ids, item_ids, concept_ids, users_w, item_resp_w,
           nb_modalities, mask):
    B = user_ids.shape[0]
    ITEM_N, M = mask.shape
    USER_N, D = users_w.shape
    item_tab = item_resp_w.reshape(ITEM_N, M * D)
    nb_f = nb_modalities.astype(jnp.float32)
    kern = _build(B, ITEM_N, USER_N, M, D, 64)
    return kern(user_ids, item_ids, users_w, item_tab, nb_f)
